# Initial kernel scaffold; baseline (speedup 1.0000x reference)
import jax
import jax.numpy as jnp
from jax.experimental import pallas as pl


def kernel(in_vec, champ_embs, item_embs):
    raise NotImplementedError("write your pallas kernel here")


# speedup vs baseline: 7.8248x; 7.8248x over previous
import functools

import jax
import jax.numpy as jnp
from jax import lax
from jax.experimental import pallas as pl
from jax.experimental.pallas import tpu as pltpu
from jax.experimental.pallas import tpu_sc as plsc

BATCH = 4096
VEC = 221
OUT = 1153
L = 16
NW = 32
RPW = BATCH // NW
NG = RPW // L

CE_PAD = 912
IE_PAD = 1760

_ZERO_OFFS = (
    tuple(range(0, 160, 16))
    + tuple(range(161, 561, 16))
    + tuple(range(567, 823, 16))
    + tuple(range(947, 1091, 16))
    + (1091,)
)


def _body(in_hbm, ce_hbm, ie_hbm, out_hbm, ce_v, ie_v, in_v, out_v):
    wid = lax.axis_index("s") * 2 + lax.axis_index("c")
    row0 = wid * RPW

    pltpu.sync_copy(ce_hbm, ce_v)
    pltpu.sync_copy(ie_hbm, ie_v)

    lanes = lax.iota(jnp.int32, L)
    rb = lanes * VEC
    ob = lanes * OUT
    onef = jnp.full((L,), 1.0, jnp.float32)
    zerof = jnp.zeros((L,), jnp.float32)

    def gi(off):
        return plsc.load_gather(in_v, [rb + off])

    def sst(idx, val):
        plsc.store_scatter(out_v, [ob + idx], val)

    def sadd(idx, val):
        plsc.addupdate_scatter(out_v, [ob + idx], val)

    def group(g, carry):
        base_row = row0 + g * L
        pltpu.sync_copy(in_hbm.at[pl.ds(base_row * VEC, L * VEC)], in_v)

        def zrow(r, c):
            b = r * OUT
            for off in _ZERO_OFFS:
                out_v[pl.ds(b + off, L)] = zerof
            return c

        lax.fori_loop(0, L, zrow, 0)

        posf = gi(0)
        posi = posf.astype(jnp.int32)
        tchi = gi(1 + posi).astype(jnp.int32)
        ochi = gi(6 + posi).astype(jnp.int32)

        sadd(posi, onef)
        sadd(5 + tchi, onef)
        sadd(411 + ochi, onef)
        for c in range(5):
            oc = gi(6 + c).astype(jnp.int32)
            sadd(947 + oc, onef)

        itb_t = 11 + 12 * posi
        itb_o = itb_t + 60
        for j in range(6):
            tid = gi(itb_t + 2 * j).astype(jnp.int32)
            tcnt = gi(itb_t + 2 * j + 1)
            sadd(161 + tid, tcnt)
            oid = gi(itb_o + 2 * j).astype(jnp.int32)
            ocnt = gi(itb_o + 2 * j + 1)
            sadd(567 + oid, ocnt)

        sst(1097, gi(211 + posi))
        sst(1098, gi(141 + posi) + gi(151 + posi))
        kb = 181 + 3 * posi
        for k in range(3):
            sst(1099 + k, gi(kb + k))
        sst(1102, gi(171 + posi))

        tce = tchi * 6
        oce = ochi * 6
        for d in range(6):
            sst(155 + d, plsc.load_gather(ce_v, [tce + d]))
            sst(561 + d, plsc.load_gather(ce_v, [oce + d]))

        def champ(c, cc):
            ci = gi(1 + c).astype(jnp.int32) * 6
            for d in range(6):
                sst(817 + 6 * c + d, plsc.load_gather(ce_v, [ci + d]))
            ibc = 11 + 12 * c
            acc = [zerof] * 7
            for j in range(6):
                iid = gi(ibc + 2 * j).astype(jnp.int32) * 7
                icnt = gi(ibc + 2 * j + 1)
                for d in range(7):
                    acc[d] = acc[d] + icnt * plsc.load_gather(ie_v, [iid + d])
            for d in range(7):
                sst(877 + 7 * c + d, acc[d])
            return cc

        lax.fori_loop(0, 10, champ, 0)

        for w in range(40):
            sst(1103 + w, gi(171 + w))
        for w in range(10):
            sst(1143 + w, gi(141 + w))

        pltpu.sync_copy(out_v, out_hbm.at[pl.ds(base_row * OUT, L * OUT)])
        return carry

    lax.fori_loop(0, NG, group, 0)


def _make_sc_call(interpret=False):
    return pl.kernel(
        _body,
        out_type=jax.ShapeDtypeStruct((BATCH * OUT,), jnp.float32),
        mesh=plsc.VectorSubcoreMesh(core_axis_name="c", subcore_axis_name="s"),
        scratch_types=[
            pltpu.VMEM((CE_PAD,), jnp.float32),
            pltpu.VMEM((IE_PAD,), jnp.float32),
            pltpu.VMEM((L * VEC,), jnp.float32),
            pltpu.VMEM((L * OUT,), jnp.float32),
        ],
        compiler_params=pltpu.CompilerParams(needs_layout_passes=False),
        interpret=interpret,
    )


@jax.jit
def kernel(in_vec, champ_embs, item_embs):
    in_flat = in_vec.reshape(-1)
    ce = jnp.pad(champ_embs.reshape(-1), (0, CE_PAD - 900))
    ie = jnp.pad(item_embs.reshape(-1), (0, IE_PAD - 1750))
    out = _make_sc_call()(in_flat, ce, ie)
    return out.reshape(BATCH, OUT)


# speedup vs baseline: 8.1567x; 1.0424x over previous
import jax
import jax.numpy as jnp
from jax import lax
from jax.experimental import pallas as pl
from jax.experimental.pallas import tpu as pltpu
from jax.experimental.pallas import tpu_sc as plsc

BATCH = 4096
VEC = 221
OUT = 1153
L = 16
NW = 32
RPW = BATCH // NW
NG = RPW // L

CE_PAD = 912
IE_PAD = 1760


def _body(in_hbm, ce_hbm, ie_hbm, out_hbm,
          ce_v, ie_v, in_v, out_v0, out_v1, sem0, sem1):
    wid = lax.axis_index("s") * 2 + lax.axis_index("c")
    row0 = wid * RPW

    pltpu.sync_copy(ce_hbm, ce_v)
    pltpu.sync_copy(ie_hbm, ie_v)
    pltpu.sync_copy(in_hbm.at[pl.ds(row0 * VEC, RPW * VEC)], in_v)

    lanes = lax.iota(jnp.int32, L)
    ob = lanes * OUT
    onef = jnp.full((L,), 1.0, jnp.float32)
    zerof = jnp.zeros((L,), jnp.float32)

    def zboth(i, c):
        out_v0[pl.ds(i * L, L)] = zerof
        out_v1[pl.ds(i * L, L)] = zerof
        return c

    lax.fori_loop(0, OUT, zboth, 0)

    def gather_scatter_ctx(g):
        return (g * L + lanes) * VEC

    def compute(g, out_v):
        rb = gather_scatter_ctx(g)

        def gi(idx):
            return plsc.load_gather(in_v, [idx])

        def sst(idx, val):
            plsc.store_scatter(out_v, [ob + idx], val)

        def sadd(idx, val):
            plsc.addupdate_scatter(out_v, [ob + idx], val)

        posi = gi(rb).astype(jnp.int32)
        rbp = rb + posi
        tchi = gi(rbp + 1).astype(jnp.int32)
        ochi = gi(rbp + 6).astype(jnp.int32)

        sadd(posi, onef)
        sadd(5 + tchi, onef)
        sadd(411 + ochi, onef)
        for c in range(5):
            oc = gi(rb + (6 + c)).astype(jnp.int32)
            sadd(947 + oc, onef)

        rbt = rb + 11 + 12 * posi
        rbo = rbt + 60
        for j in range(6):
            tid = gi(rbt + 2 * j).astype(jnp.int32)
            tcnt = gi(rbt + (2 * j + 1))
            sadd(161 + tid, tcnt)
            oid = gi(rbo + 2 * j).astype(jnp.int32)
            ocnt = gi(rbo + (2 * j + 1))
            sadd(567 + oid, ocnt)

        sst(1097, gi(rbp + 211))
        sst(1098, gi(rbp + 141) + gi(rbp + 151))
        kb = rb + 181 + 3 * posi
        for k in range(3):
            sst(1099 + k, gi(kb + k))
        sst(1102, gi(rbp + 171))

        tce = tchi * 6
        oce = ochi * 6
        for d in range(6):
            sst(155 + d, plsc.load_gather(ce_v, [tce + d]))
            sst(561 + d, plsc.load_gather(ce_v, [oce + d]))

        def champ(c, cc):
            ci = gi(rb + (1 + c)).astype(jnp.int32) * 6
            for d in range(6):
                sst(817 + 6 * c + d, plsc.load_gather(ce_v, [ci + d]))
            rbc = rb + (11 + 12 * c)
            acc = [zerof] * 7
            for j in range(6):
                iid = gi(rbc + 2 * j).astype(jnp.int32) * 7
                icnt = gi(rbc + (2 * j + 1))
                for d in range(7):
                    acc[d] = acc[d] + icnt * plsc.load_gather(ie_v, [iid + d])
            for d in range(7):
                sst(877 + 7 * c + d, acc[d])
            return cc

        lax.fori_loop(0, 10, champ, 0)

        for w in range(40):
            sst(1103 + w, gi(rb + (171 + w)))
        for w in range(10):
            sst(1143 + w, gi(rb + (141 + w)))

    def unscatter(g, out_v):
        rb = gather_scatter_ctx(g)

        def gi(idx):
            return plsc.load_gather(in_v, [idx])

        def szero(idx):
            plsc.store_scatter(out_v, [ob + idx], zerof)

        posi = gi(rb).astype(jnp.int32)
        rbp = rb + posi
        tchi = gi(rbp + 1).astype(jnp.int32)
        ochi = gi(rbp + 6).astype(jnp.int32)
        szero(posi)
        szero(5 + tchi)
        szero(411 + ochi)
        for c in range(5):
            oc = gi(rb + (6 + c)).astype(jnp.int32)
            szero(947 + oc)
        rbt = rb + 11 + 12 * posi
        rbo = rbt + 60
        for j in range(6):
            tid = gi(rbt + 2 * j).astype(jnp.int32)
            szero(161 + tid)
            oid = gi(rbo + 2 * j).astype(jnp.int32)
            szero(567 + oid)

    def start_out(g, out_v, sem):
        base = (row0 + g * L) * OUT
        pltpu.async_copy(out_v, out_hbm.at[pl.ds(base, L * OUT)], sem)

    def wait_out(out_v, sem):
        pltpu.make_async_copy(
            out_v, out_hbm.at[pl.ds(0, L * OUT)], sem).wait()

    compute(0, out_v0)
    start_out(0, out_v0, sem0)
    compute(1, out_v1)
    start_out(1, out_v1, sem1)

    def pair(k, c):
        g0 = 2 * k
        wait_out(out_v0, sem0)
        unscatter(g0 - 2, out_v0)
        compute(g0, out_v0)
        start_out(g0, out_v0, sem0)
        g1 = 2 * k + 1
        wait_out(out_v1, sem1)
        unscatter(g1 - 2, out_v1)
        compute(g1, out_v1)
        start_out(g1, out_v1, sem1)
        return c

    lax.fori_loop(1, NG // 2, pair, 0)
    wait_out(out_v0, sem0)
    wait_out(out_v1, sem1)


def _make_sc_call(interpret=False):
    return pl.kernel(
        _body,
        out_type=jax.ShapeDtypeStruct((BATCH * OUT,), jnp.float32),
        mesh=plsc.VectorSubcoreMesh(core_axis_name="c", subcore_axis_name="s"),
        scratch_types=[
            pltpu.VMEM((CE_PAD,), jnp.float32),
            pltpu.VMEM((IE_PAD,), jnp.float32),
            pltpu.VMEM((RPW * VEC,), jnp.float32),
            pltpu.VMEM((L * OUT,), jnp.float32),
            pltpu.VMEM((L * OUT,), jnp.float32),
            pltpu.SemaphoreType.DMA,
            pltpu.SemaphoreType.DMA,
        ],
        compiler_params=pltpu.CompilerParams(needs_layout_passes=False),
        interpret=interpret,
    )


@jax.jit
def kernel(in_vec, champ_embs, item_embs):
    in_flat = in_vec.reshape(-1)
    ce = jnp.pad(champ_embs.reshape(-1), (0, CE_PAD - 900))
    ie = jnp.pad(item_embs.reshape(-1), (0, IE_PAD - 1750))
    out = _make_sc_call()(in_flat, ce, ie)
    return out.reshape(BATCH, OUT)
